# baseline (speedup 1.0000x reference)
import functools

import jax
import jax.numpy as jnp
from jax import lax
from jax.experimental import pallas as pl
from jax.experimental.pallas import tpu as pltpu
from jax.experimental.pallas import tpu_sc as plsc

EMBED_DIM = 64
CTX = 20
LANES = 16


def _skipgram_sc(center_flat, ctx_flat, v_weight, u_weight, batch):
    info = plsc.get_sparse_core_info()
    nc, ns = info.num_cores, info.num_subcores
    nw = nc * ns
    per_w = batch // nw
    chunk = 64
    n_chunks = per_w // chunk
    n_gathers = (chunk * CTX) // 128

    mesh = plsc.VectorSubcoreMesh(core_axis_name="c", subcore_axis_name="s")

    @functools.partial(
        pl.kernel,
        mesh=mesh,
        compiler_params=pltpu.CompilerParams(
            needs_layout_passes=False, use_tc_tiling_on_sc=False),
        out_type=jax.ShapeDtypeStruct((batch * CTX,), jnp.float32),
        scratch_types=[
            pltpu.VMEM((chunk,), jnp.int32),
            pltpu.VMEM((chunk * CTX,), jnp.int32),
            pltpu.VMEM((chunk, EMBED_DIM), jnp.float32),
            pltpu.VMEM((chunk * CTX, EMBED_DIM), jnp.float32),
            pltpu.VMEM((chunk * CTX,), jnp.float32),
            pltpu.SemaphoreType.DMA,
        ],
    )
    def sk(center_hbm, ctx_hbm, v_hbm, u_hbm, out_hbm,
           cidx, uidx, vrows, urows, outb, sem):
        wid = lax.axis_index("s") * nc + lax.axis_index("c")

        def chunk_body(g, carry):
            base = wid * per_w + g * chunk
            pltpu.sync_copy(center_hbm.at[pl.ds(base, chunk)], cidx)
            pltpu.sync_copy(ctx_hbm.at[pl.ds(base * CTX, chunk * CTX)], uidx)
            cps = [pltpu.async_copy(v_hbm.at[cidx], vrows, sem)]
            for j in range(n_gathers):
                cps.append(pltpu.async_copy(
                    u_hbm.at[uidx.at[pl.ds(j * 128, 128)]],
                    urows.at[pl.ds(j * 128, 128)],
                    sem))
            for cp in cps:
                cp.wait()

            lane = lax.iota(jnp.int32, LANES)

            def grp_body(gi, bc):
                b0 = gi * 4
                vv = [[vrows[b0 + bb, pl.ds(k * LANES, LANES)]
                       for k in range(4)] for bb in range(4)]
                r0 = b0 * CTX
                ov = jnp.zeros((LANES,), jnp.float32)
                for r in range(4 * CTX):
                    bb, l = r // CTX, r % CTX
                    row = r0 + r
                    p = urows[row, pl.ds(0, LANES)] * vv[bb][0]
                    for k in range(1, 4):
                        p += urows[row, pl.ds(k * LANES, LANES)] * vv[bb][k]
                    s = jnp.sum(p)
                    ov = jnp.where(lane == (r % LANES), s, ov)
                    if r % LANES == LANES - 1:
                        outb[pl.ds(r0 + (r // LANES) * LANES, LANES)] = ov
                        ov = jnp.zeros((LANES,), jnp.float32)
                return bc

            lax.fori_loop(0, chunk // 4, grp_body, 0)
            pltpu.sync_copy(outb, out_hbm.at[pl.ds(base * CTX, chunk * CTX)])
            return carry

        lax.fori_loop(0, n_chunks, chunk_body, 0)

    return sk(center_flat, ctx_flat, v_weight, u_weight)


def kernel(center, contexts_and_negatives, v_weight, u_weight):
    batch = center.shape[0]
    center_flat = center.reshape(batch).astype(jnp.int32)
    ctx_flat = contexts_and_negatives.reshape(batch * CTX).astype(jnp.int32)
    out = _skipgram_sc(center_flat, ctx_flat, v_weight, u_weight, batch)
    return out.reshape(batch, 1, CTX)


# speedup vs baseline: 1.4148x; 1.4148x over previous
import functools

import jax
import jax.numpy as jnp
from jax import lax
from jax.experimental import pallas as pl
from jax.experimental.pallas import tpu as pltpu
from jax.experimental.pallas import tpu_sc as plsc

EMBED_DIM = 64
CTX = 20
LANES = 16


def _skipgram_sc(center_flat, ctx_flat, v_weight, u_weight, batch):
    info = plsc.get_sparse_core_info()
    nc, ns = info.num_cores, info.num_subcores
    nw = nc * ns
    per_w = batch // nw
    chunk = 32
    n_chunks = per_w // chunk
    nrow = chunk * CTX

    mesh = plsc.VectorSubcoreMesh(core_axis_name="c", subcore_axis_name="s")

    @functools.partial(
        pl.kernel,
        mesh=mesh,
        compiler_params=pltpu.CompilerParams(needs_layout_passes=False),
        out_type=jax.ShapeDtypeStruct((batch * CTX,), jnp.float32),
        scratch_types=[
            pltpu.VMEM((chunk,), jnp.int32),
            pltpu.VMEM((nrow,), jnp.int32),
            pltpu.VMEM((chunk, EMBED_DIM), jnp.float32),
            pltpu.VMEM((nrow, EMBED_DIM), jnp.float32),
            pltpu.VMEM((nrow,), jnp.float32),
            pltpu.SemaphoreType.DMA,
        ],
    )
    def sk(center_hbm, ctx_hbm, v_hbm, u_hbm, out_hbm,
           cidx_v, uidx_v, vrows, urows, outb, sem):
        wid = lax.axis_index("s") * nc + lax.axis_index("c")

        def chunk_body(g, carry):
            base = wid * per_w + g * chunk
            pltpu.sync_copy(center_hbm.at[pl.ds(base, chunk)], cidx_v)
            pltpu.sync_copy(ctx_hbm.at[pl.ds(base * CTX, nrow)], uidx_v)

            def fire_v(jj, bc):
                ivec = cidx_v[pl.ds(jj * LANES, LANES)]
                for k in range(LANES):
                    pltpu.async_copy(
                        v_hbm.at[ivec[k]], vrows.at[jj * LANES + k], sem)
                return bc

            def fire_u(jj, bc):
                ivec = uidx_v[pl.ds(jj * LANES, LANES)]
                for k in range(LANES):
                    pltpu.async_copy(
                        u_hbm.at[ivec[k]], urows.at[jj * LANES + k], sem)
                return bc

            lax.fori_loop(0, chunk // LANES, fire_v, 0)
            lax.fori_loop(0, nrow // LANES, fire_u, 0)
            pltpu.make_async_copy(v_hbm.at[pl.ds(0, chunk)], vrows, sem).wait()
            pltpu.make_async_copy(u_hbm.at[pl.ds(0, nrow)], urows, sem).wait()

            lane = lax.iota(jnp.int32, LANES)

            def grp_body(gi, bc):
                b0 = gi * 4
                vv = [[vrows[b0 + bb, pl.ds(k * LANES, LANES)]
                       for k in range(4)] for bb in range(4)]
                r0 = b0 * CTX
                ov = jnp.zeros((LANES,), jnp.float32)
                for r in range(4 * CTX):
                    bb = r // CTX
                    row = r0 + r
                    p = urows[row, pl.ds(0, LANES)] * vv[bb][0]
                    for k in range(1, 4):
                        p += urows[row, pl.ds(k * LANES, LANES)] * vv[bb][k]
                    s = jnp.sum(p)
                    ov = jnp.where(lane == (r % LANES), s, ov)
                    if r % LANES == LANES - 1:
                        outb[pl.ds(r0 + (r // LANES) * LANES, LANES)] = ov
                        ov = jnp.zeros((LANES,), jnp.float32)
                return bc

            lax.fori_loop(0, chunk // 4, grp_body, 0)
            pltpu.sync_copy(outb, out_hbm.at[pl.ds(base * CTX, nrow)])
            return carry

        lax.fori_loop(0, n_chunks, chunk_body, 0)

    return sk(center_flat, ctx_flat, v_weight, u_weight)


def kernel(center, contexts_and_negatives, v_weight, u_weight):
    batch = center.shape[0]
    center_flat = center.reshape(batch).astype(jnp.int32)
    ctx_flat = contexts_and_negatives.reshape(batch * CTX).astype(jnp.int32)
    out = _skipgram_sc(center_flat, ctx_flat, v_weight, u_weight, batch)
    return out.reshape(batch, 1, CTX)
